# baseline (speedup 1.0000x reference)
import functools

import jax
import jax.numpy as jnp
from jax.experimental import pallas as pl
from jax.experimental.pallas import tpu as pltpu

ROWS, COLS = 4096, 8192
R = 1024
W = 128
NC = COLS // W


def _revcumsum_kernel(x_ref, o_ref, carry_ref):
    j = pl.program_id(1)

    @pl.when(j == 0)
    def _():
        carry_ref[...] = jnp.zeros_like(carry_ref)

    xb = x_ref[...]
    s = jax.lax.broadcasted_iota(jnp.int32, (W, W), 0)
    t = jax.lax.broadcasted_iota(jnp.int32, (W, W), 1)
    tri = (s >= t).astype(jnp.float32)
    part = jax.lax.dot(xb, tri, preferred_element_type=jnp.float32)
    out = part + carry_ref[:, :1]
    o_ref[...] = out
    carry_ref[...] = out[:, :1]


@jax.jit
def kernel(x):
    grid = (ROWS // R, NC)
    return pl.pallas_call(
        _revcumsum_kernel,
        grid=grid,
        in_specs=[pl.BlockSpec((R, W), lambda i, j: (i, NC - 1 - j))],
        out_specs=pl.BlockSpec((R, W), lambda i, j: (i, NC - 1 - j)),
        out_shape=jax.ShapeDtypeStruct((ROWS, COLS), jnp.float32),
        scratch_shapes=[pltpu.VMEM((R, 1), jnp.float32)],
        compiler_params=pltpu.CompilerParams(
            dimension_semantics=("parallel", "arbitrary")
        ),
    )(x)


# speedup vs baseline: 1.9504x; 1.9504x over previous
import functools

import jax
import jax.numpy as jnp
from jax.experimental import pallas as pl
from jax.experimental.pallas import tpu as pltpu

ROWS, COLS = 4096, 8192
R = 1024
W = 512
NC = COLS // W


def _revcumsum_kernel(x_ref, o_ref, carry_ref):
    j = pl.program_id(1)

    @pl.when(j == 0)
    def _():
        carry_ref[...] = jnp.zeros_like(carry_ref)

    xb = x_ref[...]
    s = jax.lax.broadcasted_iota(jnp.int32, (W, W), 0)
    t = jax.lax.broadcasted_iota(jnp.int32, (W, W), 1)
    tri = (s >= t).astype(jnp.float32)
    part = jax.lax.dot(xb, tri, preferred_element_type=jnp.float32)
    out = part + carry_ref[:, :1]
    o_ref[...] = out
    carry_ref[...] = out[:, :1]


@jax.jit
def kernel(x):
    grid = (ROWS // R, NC)
    return pl.pallas_call(
        _revcumsum_kernel,
        grid=grid,
        in_specs=[pl.BlockSpec((R, W), lambda i, j: (i, NC - 1 - j))],
        out_specs=pl.BlockSpec((R, W), lambda i, j: (i, NC - 1 - j)),
        out_shape=jax.ShapeDtypeStruct((ROWS, COLS), jnp.float32),
        scratch_shapes=[pltpu.VMEM((R, 1), jnp.float32)],
        compiler_params=pltpu.CompilerParams(
            dimension_semantics=("parallel", "arbitrary")
        ),
    )(x)


# speedup vs baseline: 2.0428x; 1.0474x over previous
import functools

import jax
import jax.numpy as jnp
from jax.experimental import pallas as pl
from jax.experimental.pallas import tpu as pltpu

ROWS, COLS = 4096, 8192
R = 1024
W = 1024
NC = COLS // W


def _revcumsum_kernel(x_ref, o_ref, carry_ref):
    j = pl.program_id(1)

    @pl.when(j == 0)
    def _():
        carry_ref[...] = jnp.zeros_like(carry_ref)

    xb = x_ref[...]
    s = jax.lax.broadcasted_iota(jnp.int32, (W, W), 0)
    t = jax.lax.broadcasted_iota(jnp.int32, (W, W), 1)
    tri = (s >= t).astype(jnp.float32)
    part = jax.lax.dot(xb, tri, preferred_element_type=jnp.float32)
    out = part + carry_ref[:, :1]
    o_ref[...] = out
    carry_ref[...] = out[:, :1]


@jax.jit
def kernel(x):
    grid = (ROWS // R, NC)
    return pl.pallas_call(
        _revcumsum_kernel,
        grid=grid,
        in_specs=[pl.BlockSpec((R, W), lambda i, j: (i, NC - 1 - j))],
        out_specs=pl.BlockSpec((R, W), lambda i, j: (i, NC - 1 - j)),
        out_shape=jax.ShapeDtypeStruct((ROWS, COLS), jnp.float32),
        scratch_shapes=[pltpu.VMEM((R, 1), jnp.float32)],
        compiler_params=pltpu.CompilerParams(
            dimension_semantics=("parallel", "arbitrary")
        ),
    )(x)


# speedup vs baseline: 2.4855x; 1.2167x over previous
import jax
import jax.numpy as jnp
from jax.experimental import pallas as pl
from jax.experimental.pallas import tpu as pltpu

ROWS, COLS = 4096, 8192
R = 1024
W = 2048
SUB = 256
NC = COLS // W
K = W // SUB


def _revcumsum_kernel(x_ref, o_ref, carry_ref):
    j = pl.program_id(1)

    @pl.when(j == 0)
    def _():
        carry_ref[...] = jnp.zeros_like(carry_ref)

    s = jax.lax.broadcasted_iota(jnp.int32, (SUB, SUB), 0)
    t = jax.lax.broadcasted_iota(jnp.int32, (SUB, SUB), 1)
    tri = (s >= t).astype(jnp.float32)

    carry = carry_ref[:, :1]
    for k in range(K - 1, -1, -1):
        xs = x_ref[:, k * SUB:(k + 1) * SUB]
        p = jax.lax.dot(xs, tri, preferred_element_type=jnp.float32)
        o_ref[:, k * SUB:(k + 1) * SUB] = p + carry
        carry = carry + p[:, :1]
    carry_ref[...] = carry


@jax.jit
def kernel(x):
    grid = (ROWS // R, NC)
    return pl.pallas_call(
        _revcumsum_kernel,
        grid=grid,
        in_specs=[pl.BlockSpec((R, W), lambda i, j: (i, NC - 1 - j))],
        out_specs=pl.BlockSpec((R, W), lambda i, j: (i, NC - 1 - j)),
        out_shape=jax.ShapeDtypeStruct((ROWS, COLS), jnp.float32),
        scratch_shapes=[pltpu.VMEM((R, 1), jnp.float32)],
        compiler_params=pltpu.CompilerParams(
            dimension_semantics=("parallel", "arbitrary")
        ),
    )(x)
